# baseline (speedup 1.0000x reference)
import jax
import jax.numpy as jnp
from jax.experimental import pallas as pl
from jax.experimental.pallas import tpu as pltpu

B = 64
C = 768
H = 14
W = 14
HW = H * W
FREQ = 256
E = 64
K = 8
NCORES = 2
PER_CORE = HW // NCORES
SW = 7
NSTEPS = PER_CORE // (2 * SW)


def _pool_body(x_ref, x2_ref, part_ref, acc_ref):
    g = pl.program_id(1)

    @pl.when(g == 0)
    def _init():
        acc_ref[...] = jnp.zeros_like(acc_ref)

    acc_ref[...] += jnp.sum(x_ref[...], axis=0) + jnp.sum(x2_ref[...], axis=0)

    @pl.when(g == NSTEPS - 1)
    def _done():
        part_ref[...] = acc_ref[...][None]


def _epilogue_body(part_ref, freq_ref, wg_ref, wf_ref,
                   gates_ref, idx_ref, val_ref):
    pooled = (part_ref[0] + part_ref[1]) * (1.0 / HW)
    logits = jax.lax.dot_general(
        pooled, wg_ref[...],
        dimension_numbers=(((1,), (1,)), ((), ())),
        preferred_element_type=jnp.float32,
    )
    logits += jax.lax.dot_general(
        freq_ref[...], wf_ref[...],
        dimension_numbers=(((1,), (1,)), ((), ())),
        preferred_element_type=jnp.float32,
    )

    m = jnp.max(logits, axis=-1, keepdims=True)
    ex = jnp.exp(logits - m)
    scores = ex / jnp.sum(ex, axis=-1, keepdims=True)

    iota = jax.lax.broadcasted_iota(jnp.int32, (B, E), 1)
    active = jnp.ones((B, E), dtype=jnp.bool_)
    gates = jnp.zeros((B, E), dtype=jnp.float32)
    idxs = []
    vals = []
    for _ in range(K):
        masked = jnp.where(active, scores, -jnp.inf)
        v = jnp.max(masked, axis=-1, keepdims=True)
        cand = jnp.where(masked == v, iota, E)
        i = jnp.min(cand, axis=-1, keepdims=True)
        gates = jnp.where(iota == i, v, gates)
        active = active & (iota != i)
        idxs.append(i)
        vals.append(v)

    gates_ref[...] = gates
    idx_ref[...] = jnp.concatenate(idxs, axis=-1)
    val_ref[...] = jnp.concatenate(vals, axis=-1)


@jax.jit
def kernel(x, freq_emb, W_gate, W_freq):
    xt = jnp.transpose(x, (2, 3, 0, 1)).reshape(HW, B, C)
    partials = pl.pallas_call(
        _pool_body,
        grid=(NCORES, NSTEPS),
        in_specs=[
            pl.BlockSpec((SW, B, C),
                         lambda c, g: (c * (PER_CORE // SW) + 2 * g, 0, 0)),
            pl.BlockSpec((SW, B, C),
                         lambda c, g: (c * (PER_CORE // SW) + 2 * g + 1, 0, 0)),
        ],
        out_specs=pl.BlockSpec((1, B, C), lambda c, g: (c, 0, 0)),
        out_shape=jax.ShapeDtypeStruct((NCORES, B, C), jnp.float32),
        scratch_shapes=[pltpu.VMEM((B, C), jnp.float32)],
        compiler_params=pltpu.CompilerParams(
            dimension_semantics=("parallel", "arbitrary")),
    )(xt, xt)

    gates, idx, val = pl.pallas_call(
        _epilogue_body,
        in_specs=[
            pl.BlockSpec((NCORES, B, C), lambda: (0, 0, 0)),
            pl.BlockSpec((B, FREQ), lambda: (0, 0)),
            pl.BlockSpec((E, C), lambda: (0, 0)),
            pl.BlockSpec((E, FREQ), lambda: (0, 0)),
        ],
        out_specs=[
            pl.BlockSpec((B, E), lambda: (0, 0)),
            pl.BlockSpec((B, K), lambda: (0, 0)),
            pl.BlockSpec((B, K), lambda: (0, 0)),
        ],
        out_shape=[
            jax.ShapeDtypeStruct((B, E), jnp.float32),
            jax.ShapeDtypeStruct((B, K), jnp.int32),
            jax.ShapeDtypeStruct((B, K), jnp.float32),
        ],
    )(partials, freq_emb, W_gate, W_freq)
    return gates, idx, val


# speedup vs baseline: 1.2385x; 1.2385x over previous
import jax
import jax.numpy as jnp
from jax.experimental import pallas as pl
from jax.experimental.pallas import tpu as pltpu

B = 64
C = 768
H = 14
W = 14
HW = H * W
FREQ = 256
E = 64
K = 8
NWIN = 4
SW = 7
NSTEPS = HW // (NWIN * SW)


def _routing_body(x_ref, x2_ref, x3_ref, x4_ref, freq_ref, wg_ref, wf_ref,
                  gates_ref, idx_ref, val_ref, acc_ref):
    g = pl.program_id(0)

    @pl.when(g == 0)
    def _init():
        acc_ref[...] = jnp.zeros_like(acc_ref)

    acc_ref[...] += ((jnp.sum(x_ref[...], axis=0) + jnp.sum(x2_ref[...], axis=0))
                     + (jnp.sum(x3_ref[...], axis=0) + jnp.sum(x4_ref[...], axis=0)))

    @pl.when(g == NSTEPS - 1)
    def _epilogue():
        pooled = acc_ref[...] * (1.0 / HW)
        logits = jax.lax.dot_general(
            pooled, wg_ref[...],
            dimension_numbers=(((1,), (1,)), ((), ())),
            preferred_element_type=jnp.float32,
        )
        logits += jax.lax.dot_general(
            freq_ref[...], wf_ref[...],
            dimension_numbers=(((1,), (1,)), ((), ())),
            preferred_element_type=jnp.float32,
        )

        m = jnp.max(logits, axis=-1, keepdims=True)
        ex = jnp.exp(logits - m)
        scores = ex / jnp.sum(ex, axis=-1, keepdims=True)

        iota = jax.lax.broadcasted_iota(jnp.int32, (B, E), 1)
        active = jnp.ones((B, E), dtype=jnp.bool_)
        gates = jnp.zeros((B, E), dtype=jnp.float32)
        idxs = []
        vals = []
        for _ in range(K):
            masked = jnp.where(active, scores, -jnp.inf)
            v = jnp.max(masked, axis=-1, keepdims=True)
            cand = jnp.where(masked == v, iota, E)
            i = jnp.min(cand, axis=-1, keepdims=True)
            gates = jnp.where(iota == i, v, gates)
            active = active & (iota != i)
            idxs.append(i)
            vals.append(v)

        gates_ref[...] = gates
        idx_ref[...] = jnp.concatenate(idxs, axis=-1)
        val_ref[...] = jnp.concatenate(vals, axis=-1)


@jax.jit
def kernel(x, freq_emb, W_gate, W_freq):
    xt = jnp.transpose(x, (2, 3, 0, 1)).reshape(HW, B, C)
    gates, idx, val = pl.pallas_call(
        _routing_body,
        grid=(NSTEPS,),
        in_specs=[
            pl.BlockSpec((SW, B, C), lambda g: (4 * g, 0, 0)),
            pl.BlockSpec((SW, B, C), lambda g: (4 * g + 1, 0, 0)),
            pl.BlockSpec((SW, B, C), lambda g: (4 * g + 2, 0, 0)),
            pl.BlockSpec((SW, B, C), lambda g: (4 * g + 3, 0, 0)),
            pl.BlockSpec((B, FREQ), lambda g: (0, 0)),
            pl.BlockSpec((E, C), lambda g: (0, 0)),
            pl.BlockSpec((E, FREQ), lambda g: (0, 0)),
        ],
        out_specs=[
            pl.BlockSpec((B, E), lambda g: (0, 0)),
            pl.BlockSpec((B, K), lambda g: (0, 0)),
            pl.BlockSpec((B, K), lambda g: (0, 0)),
        ],
        out_shape=[
            jax.ShapeDtypeStruct((B, E), jnp.float32),
            jax.ShapeDtypeStruct((B, K), jnp.int32),
            jax.ShapeDtypeStruct((B, K), jnp.float32),
        ],
        scratch_shapes=[pltpu.VMEM((B, C), jnp.float32)],
    )(xt, xt, xt, xt, freq_emb, W_gate, W_freq)
    return gates, idx, val
